# baseline (speedup 1.0000x reference)
import dataclasses

import jax
import jax.numpy as jnp
from jax import lax
from jax.experimental import pallas as pl
from jax.experimental.pallas import tpu as pltpu
from jax.experimental.pallas import tpu_sc as plsc

DIM = 1024
D32 = DIM // 2
INTER = 1024
NR = 15
TOPK = 7
K8 = 8
S = 2048
LANES = 128
BM = 256
NBLK = 71
PADTOT = NBLK * BM
NC, NS = 2, 16
NW = NC * NS
TOK_PER_W = S // NW
CH = 16
NIDX = S * K8
ROWS_PER_W = NIDX // NW
GCH = 64
SQRT1_2 = 0.7071067811865476


def _gelu(h):
    return 0.5 * h * (1.0 + jax.lax.erf(h * SQRT1_2))


def _cumsum0(a):
    n, w = a.shape
    sh = 1
    while sh < n:
        z = jnp.zeros((sh, w), a.dtype)
        a = a + jnp.concatenate([z, a[:-sh]], axis=0)
        sh *= 2
    return a


def _router_kernel(x_ref, wr_ref, br_ref,
                   g_ref, dst_ref, cnt_ref, xbf_ref):
    x = x_ref[...]
    xbf_ref[...] = x.astype(jnp.bfloat16)
    logits = jnp.dot(x, wr_ref[...], preferred_element_type=jnp.float32)
    logits = logits + br_ref[...]
    lane = jax.lax.broadcasted_iota(jnp.int32, logits.shape, 1)
    valid = lane < NR
    logits = jnp.where(valid, logits, -1e30)
    m = jnp.max(logits, axis=1, keepdims=True)
    ex = jnp.where(valid, jnp.exp(logits - m), 0.0)
    aff = ex / jnp.sum(ex, axis=1, keepdims=True)

    work = aff
    sels = []
    for _ in range(TOPK):
        idx = jnp.argmax(work, axis=1)
        sel = lane == idx[:, None]
        sels.append(sel)
        work = jnp.where(sel, -1.0, work)
    msel = jnp.zeros_like(aff)
    for sel in sels:
        msel = jnp.where(sel, 1.0, msel)

    rank = _cumsum0(msel) - msel
    counts = jnp.sum(msel, axis=0, keepdims=True)
    cnt_ref[...] = counts
    cpad = jnp.floor((counts + (BM - 1)) * (1.0 / BM)) * BM
    elane = jax.lax.broadcasted_iota(jnp.int32, (LANES, LANES), 0)
    flane = jax.lax.broadcasted_iota(jnp.int32, (LANES, LANES), 1)
    ltm = jnp.where(elane < flane, 1.0, 0.0)
    off = jnp.dot(cpad, ltm, preferred_element_type=jnp.float32)
    pos = off + rank

    k8lane = jax.lax.broadcasted_iota(jnp.int32, (S, K8), 1)
    gv = jnp.zeros((S, K8), jnp.float32)
    dst = jnp.zeros((S, K8), jnp.float32)
    for k, sel in enumerate(sels):
        gk = jnp.sum(jnp.where(sel, aff, 0.0), axis=1, keepdims=True)
        pk = jnp.sum(jnp.where(sel, pos, 0.0), axis=1, keepdims=True)
        gv = jnp.where(k8lane == k, gk, gv)
        dst = jnp.where(k8lane == k, pk, dst)
    g_ref[...] = gv
    dst_ref[...] = dst.astype(jnp.int32)


def _shared_kernel(x_ref, w1s_ref, b1s_ref, w2s_ref, b2s_ref, oinit_ref):
    x = x_ref[...]
    xb = x.astype(jnp.bfloat16)
    h = jnp.dot(xb, w1s_ref[...].astype(jnp.bfloat16),
                preferred_element_type=jnp.float32) + b1s_ref[...]
    h = _gelu(h)
    eo = jnp.dot(h.astype(jnp.bfloat16), w2s_ref[...].astype(jnp.bfloat16),
                 preferred_element_type=jnp.float32) + b2s_ref[...]
    oinit_ref[...] = x + eo


def _sc_scatter_kernel(xb_hbm, dst_hbm, xs_hbm, xrow_v, dst_v, sem):
    wid = lax.axis_index("s") * NC + lax.axis_index("c")
    base = wid * TOK_PER_W
    pltpu.sync_copy(xb_hbm.at[pl.ds(base, TOK_PER_W)], xrow_v)
    pltpu.sync_copy(dst_hbm.at[pl.ds(base, TOK_PER_W)], dst_v)

    @pl.loop(0, TOK_PER_W, step=CH)
    def _(t0):
        cps = []
        for k in range(TOPK):
            rows = t0 + jax.lax.iota(jnp.int32, CH)
            cols = jnp.full((CH,), k, jnp.int32)
            idxv = plsc.load_gather(dst_v, [rows, cols])
            cp = pltpu.make_async_copy(
                xrow_v.at[pl.ds(t0, CH)], xs_hbm.at[idxv], sem)
            cp.start()
            cps.append(cp)
        for cp in cps:
            cp.wait()


def _sc_gather_kernel(y_hbm, dstf_hbm, yun_hbm, idx_v, rows_v, sem):
    wid = lax.axis_index("s") * NC + lax.axis_index("c")
    base = wid * ROWS_PER_W

    @pl.loop(0, ROWS_PER_W, step=GCH)
    def _(r0):
        pltpu.sync_copy(dstf_hbm.at[pl.ds(base + r0, GCH)], idx_v)
        pltpu.async_copy(y_hbm.at[idx_v], rows_v, sem).wait()
        pltpu.sync_copy(rows_v, yun_hbm.at[pl.ds(base + r0, GCH)])


def _ffn_kernel(bexp_ref, xs_ref, w1_ref, b1_ref, w2_ref, b2_ref, y_ref):
    be = bexp_ref[pl.program_id(0)]

    @pl.when(be < NR)
    def _active():
        h = jnp.dot(xs_ref[...], w1_ref[0].astype(jnp.bfloat16),
                    preferred_element_type=jnp.float32) + b1_ref[0]
        h = _gelu(h)
        eo = jnp.dot(h.astype(jnp.bfloat16), w2_ref[0].astype(jnp.bfloat16),
                     preferred_element_type=jnp.float32) + b2_ref[0]
        y_ref[...] = eo.astype(jnp.bfloat16)


def _combine_kernel(oi_ref, yun_ref, gv_ref, out_ref):
    bmc = oi_ref.shape[0]
    acc = oi_ref[...]
    y3 = yun_ref[...].reshape(bmc, K8, DIM)
    for k in range(TOPK):
        gk = gv_ref[:, k][:, None]
        acc = acc + gk * y3[:, k, :].astype(jnp.float32)
    out_ref[...] = acc


def _bf16_as_i32(a):
    return jax.lax.bitcast_convert_type(
        a.reshape(a.shape[0], a.shape[1] // 2, 2), jnp.int32)


def _i32_as_bf16(a):
    b = jax.lax.bitcast_convert_type(a, jnp.bfloat16)
    return b.reshape(a.shape[0], a.shape[1] * 2)


def kernel(x, W1s, b1s, W2s, b2s, W1r, b1r, W2r, b2r, Wr, br):
    B, _, D = x.shape
    x2 = x.reshape(S, D)

    wr_pad = jnp.zeros((D, LANES), jnp.float32).at[:, :NR].set(Wr)
    br_pad = jnp.zeros((1, LANES), jnp.float32).at[0, :NR].set(br)

    gv, dst, counts, xbf = pl.pallas_call(
        _router_kernel,
        out_shape=(
            jax.ShapeDtypeStruct((S, K8), jnp.float32),
            jax.ShapeDtypeStruct((S, K8), jnp.int32),
            jax.ShapeDtypeStruct((1, LANES), jnp.float32),
            jax.ShapeDtypeStruct((S, D), jnp.bfloat16),
        ),
    )(x2, wr_pad, br_pad)

    out_init = pl.pallas_call(
        _shared_kernel,
        out_shape=jax.ShapeDtypeStruct((S, D), jnp.float32),
    )(x2, W1s, b1s.reshape(1, INTER), W2s, b2s.reshape(1, D))

    cnt = counts[0, :NR]
    cpad_blks = jnp.ceil(cnt / BM).astype(jnp.int32)
    obl = jnp.concatenate([jnp.zeros((1,), jnp.int32),
                           jnp.cumsum(cpad_blks)])
    bexp = (jnp.sum(jnp.arange(NBLK, dtype=jnp.int32)[:, None] >= obl[None, :],
                    axis=1) - 1).astype(jnp.int32)

    xb32 = _bf16_as_i32(xbf)

    mesh = plsc.VectorSubcoreMesh(core_axis_name="c", subcore_axis_name="s")
    sc_cp = pltpu.CompilerParams()
    if "needs_layout_passes" in pltpu.CompilerParams.__dataclass_fields__:
        sc_cp = dataclasses.replace(sc_cp, needs_layout_passes=False)

    sc_scatter = pl.kernel(
        _sc_scatter_kernel,
        out_type=jax.ShapeDtypeStruct((PADTOT, D32), jnp.int32),
        mesh=mesh,
        scratch_types=[
            pltpu.VMEM((TOK_PER_W, D32), jnp.int32),
            pltpu.VMEM((TOK_PER_W, K8), jnp.int32),
            pltpu.SemaphoreType.DMA,
        ],
        compiler_params=sc_cp,
    )
    xs32 = sc_scatter(xb32, dst)

    xs_bf = _i32_as_bf16(xs32)

    y_bf = pl.pallas_call(
        _ffn_kernel,
        grid_spec=pltpu.PrefetchScalarGridSpec(
            num_scalar_prefetch=1,
            grid=(NBLK,),
            in_specs=[
                pl.BlockSpec((BM, D), lambda b, be: (b, 0)),
                pl.BlockSpec((1, D, INTER),
                             lambda b, be: (jnp.minimum(be[b], NR - 1), 0, 0)),
                pl.BlockSpec((1, 1, INTER),
                             lambda b, be: (jnp.minimum(be[b], NR - 1), 0, 0)),
                pl.BlockSpec((1, INTER, D),
                             lambda b, be: (jnp.minimum(be[b], NR - 1), 0, 0)),
                pl.BlockSpec((1, 1, D),
                             lambda b, be: (jnp.minimum(be[b], NR - 1), 0, 0)),
            ],
            out_specs=pl.BlockSpec((BM, D), lambda b, be: (b, 0)),
        ),
        out_shape=jax.ShapeDtypeStruct((PADTOT, D), jnp.bfloat16),
        compiler_params=pltpu.CompilerParams(
            dimension_semantics=("arbitrary",),
        ),
    )(bexp, xs_bf, W1r, b1r.reshape(NR, 1, INTER), W2r, b2r.reshape(NR, 1, D))

    y32 = _bf16_as_i32(y_bf)
    dstf = dst.reshape(NIDX)

    sc_gather = pl.kernel(
        _sc_gather_kernel,
        out_type=jax.ShapeDtypeStruct((NIDX, D32), jnp.int32),
        mesh=mesh,
        scratch_types=[
            pltpu.VMEM((GCH,), jnp.int32),
            pltpu.VMEM((GCH, D32), jnp.int32),
            pltpu.SemaphoreType.DMA,
        ],
        compiler_params=sc_cp,
    )
    yun32 = sc_gather(y32, dstf)
    yun_bf = _i32_as_bf16(yun32)

    BMC = 256
    out = pl.pallas_call(
        _combine_kernel,
        grid=(S // BMC,),
        in_specs=[
            pl.BlockSpec((BMC, D), lambda b: (b, 0)),
            pl.BlockSpec((BMC * K8, D), lambda b: (b, 0)),
            pl.BlockSpec((BMC, K8), lambda b: (b, 0)),
        ],
        out_specs=pl.BlockSpec((BMC, D), lambda b: (b, 0)),
        out_shape=jax.ShapeDtypeStruct((S, D), jnp.float32),
    )(out_init, yun_bf, gv)

    return out.reshape(B, S, D)


# speedup vs baseline: 7.5460x; 7.5460x over previous
import jax
import jax.numpy as jnp
from jax.experimental import pallas as pl
from jax.experimental.pallas import tpu as pltpu

DIM = 1024
INTER = 1024
NR = 15
TOPK = 7
LANES = 128
SQRT1_2 = 0.7071067811865476


def _gelu(h):
    return 0.5 * h * (1.0 + jax.lax.erf(h * SQRT1_2))


def _router_kernel(x_ref, wr_ref, br_ref, w1s_ref, b1s_ref, w2s_ref, b2s_ref,
                   g_ref, xbf_ref, oinit_ref):
    x = x_ref[...]
    logits = jnp.dot(x, wr_ref[...], preferred_element_type=jnp.float32)
    logits = logits + br_ref[...]
    lane = jax.lax.broadcasted_iota(jnp.int32, logits.shape, 1)
    valid = lane < NR
    logits = jnp.where(valid, logits, -1e30)
    m = jnp.max(logits, axis=1, keepdims=True)
    ex = jnp.where(valid, jnp.exp(logits - m), 0.0)
    aff = ex / jnp.sum(ex, axis=1, keepdims=True)
    work = aff
    gates = jnp.zeros_like(aff)
    for _ in range(TOPK):
        idx = jnp.argmax(work, axis=1)
        sel = lane == idx[:, None]
        gates = jnp.where(sel, aff, gates)
        work = jnp.where(sel, -1.0, work)
    g_ref[...] = gates

    xb = x.astype(jnp.bfloat16)
    xbf_ref[...] = xb
    h = jnp.dot(xb, w1s_ref[...].astype(jnp.bfloat16),
                preferred_element_type=jnp.float32) + b1s_ref[...]
    h = _gelu(h)
    eo = jnp.dot(h.astype(jnp.bfloat16), w2s_ref[...].astype(jnp.bfloat16),
                 preferred_element_type=jnp.float32) + b2s_ref[...]
    oinit_ref[...] = x + eo


def _expert_kernel(oi_ref, g_ref, xbf_ref, w1_ref, b1_ref, w2_ref, b2_ref,
                   out_ref):
    e = pl.program_id(0)
    h = jnp.dot(xbf_ref[...], w1_ref[0].astype(jnp.bfloat16),
                preferred_element_type=jnp.float32) + b1_ref[0]
    h = _gelu(h)
    eo = jnp.dot(h.astype(jnp.bfloat16), w2_ref[0].astype(jnp.bfloat16),
                 preferred_element_type=jnp.float32) + b2_ref[0]
    lane = jax.lax.broadcasted_iota(jnp.int32, g_ref.shape, 1)
    g = jnp.sum(jnp.where(lane == e, g_ref[...], 0.0), axis=1, keepdims=True)
    contrib = eo * g

    @pl.when(e == 0)
    def _first():
        out_ref[...] = oi_ref[...] + contrib

    @pl.when(e != 0)
    def _rest():
        out_ref[...] += contrib


def kernel(x, W1s, b1s, W2s, b2s, W1r, b1r, W2r, b2r, Wr, br):
    B, S, D = x.shape
    x2 = x.reshape(S, D)

    wr_pad = jnp.zeros((D, LANES), jnp.float32).at[:, :NR].set(Wr)
    br_pad = jnp.zeros((1, LANES), jnp.float32).at[0, :NR].set(br)

    gates, xbf, out_init = pl.pallas_call(
        _router_kernel,
        out_shape=(
            jax.ShapeDtypeStruct((S, LANES), jnp.float32),
            jax.ShapeDtypeStruct((S, D), jnp.bfloat16),
            jax.ShapeDtypeStruct((S, D), jnp.float32),
        ),
    )(x2, wr_pad, br_pad, W1s, b1s.reshape(1, INTER), W2s, b2s.reshape(1, D))

    out = pl.pallas_call(
        _expert_kernel,
        grid=(NR,),
        in_specs=[
            pl.BlockSpec((S, D), lambda e: (0, 0)),
            pl.BlockSpec((S, LANES), lambda e: (0, 0)),
            pl.BlockSpec((S, D), lambda e: (0, 0)),
            pl.BlockSpec((1, D, INTER), lambda e: (e, 0, 0)),
            pl.BlockSpec((1, 1, INTER), lambda e: (e, 0, 0)),
            pl.BlockSpec((1, INTER, D), lambda e: (e, 0, 0)),
            pl.BlockSpec((1, 1, D), lambda e: (e, 0, 0)),
        ],
        out_specs=pl.BlockSpec((S, D), lambda e: (0, 0)),
        out_shape=jax.ShapeDtypeStruct((S, D), jnp.float32),
        compiler_params=pltpu.CompilerParams(
            dimension_semantics=("arbitrary",),
        ),
    )(out_init, gates, xbf, W1r, b1r.reshape(NR, 1, INTER),
      W2r, b2r.reshape(NR, 1, D))

    return out.reshape(B, S, D)
